# baseline (speedup 1.0000x reference)
import dataclasses
import functools

import jax
import jax.numpy as jnp
from jax import lax
from jax.experimental import pallas as pl
from jax.experimental.pallas import tpu as pltpu
from jax.experimental.pallas import tpu_sc as plsc

N = 10000
D = 128
DP = D // 2
E = 320000
NCLS = 40

NC = 2
NS = 16
NW = NC * NS
L = 16

EPAD = 327680
EPT = EPAD // NW
C = 80
NWIN = EPT // C
RB = 4
SB = 2
IB = 8
NPAD = 10240
RPT = NPAD // NS


def _pack_tc(hw):
    u = jax.lax.bitcast_convert_type(
        hw.astype(jnp.bfloat16), jnp.uint16).astype(jnp.uint32)
    pk = [u[:, c * 32:c * 32 + 16] | (u[:, c * 32 + 16:c * 32 + 32] << 16)
          for c in range(D // 32)]
    return jnp.concatenate(pk, axis=1).astype(jnp.int32)


def _encode(x, W_enc, b_enc, W1):
    def body(x_ref, we_ref, be_ref, w1_ref, o_ref):
        h = jnp.tanh(
            jnp.dot(x_ref[...], we_ref[...], preferred_element_type=jnp.float32)
            + be_ref[...]
        )
        o_ref[...] = _pack_tc(
            jnp.dot(h, w1_ref[...], preferred_element_type=jnp.float32))

    return pl.pallas_call(
        body,
        out_shape=jax.ShapeDtypeStruct((N, DP), jnp.int32),
    )(x, W_enc, b_enc.reshape(1, D), W1)


def _mid(parts, b, W):
    def body(p_ref, b_ref, w_ref, o_ref):
        h = jnp.tanh(p_ref[0, :N, :] + p_ref[1, :N, :] + b_ref[...])
        o_ref[...] = _pack_tc(
            jnp.dot(h, w_ref[...], preferred_element_type=jnp.float32))

    return pl.pallas_call(
        body,
        out_shape=jax.ShapeDtypeStruct((N, DP), jnp.int32),
    )(parts, b.reshape(1, D), W)


def _decode(parts, b2, W_dec, b_dec):
    def body(p_ref, b2_ref, wd_ref, bd_ref, o_ref):
        h = jnp.tanh(p_ref[0, :N, :] + p_ref[1, :N, :] + b2_ref[...])
        o_ref[...] = (
            jnp.dot(h, wd_ref[...], preferred_element_type=jnp.float32)
            + bd_ref[...]
        )

    return pl.pallas_call(
        body,
        out_shape=jax.ShapeDtypeStruct((N, NCLS), jnp.float32),
    )(parts, b2.reshape(1, D), W_dec, b_dec.reshape(1, NCLS))


def _sc_edge_pass(hwp, e3):
    mesh = plsc.VectorSubcoreMesh(core_axis_name="c", subcore_axis_name="s")
    cp = pltpu.CompilerParams()
    if "needs_layout_passes" in pltpu.CompilerParams.__dataclass_fields__:
        cp = dataclasses.replace(cp, needs_layout_passes=False)
    if "use_tc_tiling_on_sc" in pltpu.CompilerParams.__dataclass_fields__:
        cp = dataclasses.replace(cp, use_tc_tiling_on_sc=False)

    @functools.partial(
        pl.kernel,
        mesh=mesh,
        compiler_params=cp,
        out_type=jax.ShapeDtypeStruct((NC, NPAD, D), jnp.float32),
        scratch_types=(
            [pltpu.VMEM((3, C), jnp.int32) for _ in range(IB)]
            + [pltpu.VMEM((C, DP), jnp.int32) for _ in range(RB)]
            + [pltpu.VMEM((C, D), jnp.float32) for _ in range(SB)]
            + [pltpu.VMEM_SHARED((NPAD, D), jnp.float32)]
            + [pltpu.SemaphoreType.DMA for _ in range(IB + RB + SB)]
        ),
    )
    def k(hw_hbm, e3_hbm, out_hbm, *refs):
        eib = refs[0:IB]
        rows = refs[IB:IB + RB]
        stage = refs[IB + RB:IB + RB + SB]
        acc_sh = refs[IB + RB + SB]
        sems = refs[IB + RB + SB + 1:]
        isem = sems[0:IB]
        gsem = sems[IB:IB + RB]
        ssem = sems[IB + RB:IB + RB + SB]

        cid = lax.axis_index("c")
        sid = lax.axis_index("s")
        wbase = (cid * NS + sid) * NWIN

        def idx_issue(wi, j):
            pltpu.async_copy(e3_hbm.at[wbase + wi], eib[j], isem[j])

        def idx_wait(wi, j):
            pltpu.make_async_copy(
                e3_hbm.at[wbase + wi], eib[j], isem[j]).wait()

        def gather_issue(j, b):
            pltpu.async_copy(hw_hbm.at[eib[j].at[0]], rows[b], gsem[b])

        def gather_wait(j, b):
            pltpu.make_async_copy(
                hw_hbm.at[eib[j].at[0]], rows[b], gsem[b]).wait()

        def scatter_issue(j, sb):
            pltpu.async_copy(stage[sb], acc_sh.at[eib[j].at[1]], ssem[sb],
                             add=True)

        def scatter_wait(j, sb):
            pltpu.make_async_copy(
                stage[sb], acc_sh.at[eib[j].at[1]], ssem[sb]).wait()

        for j in range(5):
            idx_issue(j, j)
        for j in range(3):
            idx_wait(j, j)
            gather_issue(j, j)

        @pl.loop(0, C)
        def _(r):
            for cc in range(D // L):
                stage[0][r, pl.ds(cc * L, L)] = jnp.zeros((L,), jnp.float32)
        for t in range(RPT // C):
            pltpu.sync_copy(stage[0],
                            acc_sh.at[pl.ds(sid * RPT + t * C, C)])
        plsc.subcore_barrier()

        @pl.loop(0, NWIN, step=IB)
        def _(w):
            for b in range(IB):
                wi = w + b
                rb = b % RB
                rb3 = (b + 3) % RB
                sb = b % SB
                j3 = (b + 3) % IB
                j5 = (b + 5) % IB

                @pl.when(wi + 5 < NWIN)
                def _():
                    idx_issue(wi + 5, j5)

                @pl.when(wi + 3 < NWIN)
                def _():
                    idx_wait(wi + 3, j3)
                    gather_issue(j3, rb3)

                gather_wait(b, rb)

                @pl.when(wi >= SB)
                def _():
                    scatter_wait((b - SB) % IB, sb)

                @plsc.parallel_loop(0, C, unroll=2)
                def _(r):
                    wvec = plsc.bitcast(plsc.load_gather(
                        eib[b],
                        [jnp.full((L,), 2, jnp.int32),
                         jnp.full((L,), r, jnp.int32)]), jnp.float32)
                    for cc in range(D // 32):
                        pk = rows[rb][r, pl.ds(cc * 16, 16)]
                        bfv = plsc.bitcast(pk, jnp.bfloat16)
                        lo, hi = plsc.unpack(
                            bfv, format=plsc.PackFormat.INTERLEAVED)
                        stage[sb][r, pl.ds(cc * 32, L)] = lo * wvec
                        stage[sb][r, pl.ds(cc * 32 + L, L)] = hi * wvec

                scatter_issue(b, sb)

        for s in range(SB):
            scatter_wait((NWIN - SB + s) % IB, (NWIN - SB + s) % SB)

        plsc.subcore_barrier()
        pltpu.sync_copy(acc_sh.at[pl.ds(sid * RPT, RPT)],
                        out_hbm.at[cid, pl.ds(sid * RPT, RPT)])

    return k(hwp, e3)


def kernel(x, edge_index, edge_weight, W_enc, b_enc, W1, b1, W2, b2, W_dec, b_dec):
    pad = EPAD - E
    src = jnp.concatenate(
        [edge_index[0].astype(jnp.int32), jnp.zeros((pad,), jnp.int32)])
    dst = jnp.concatenate(
        [edge_index[1].astype(jnp.int32), jnp.zeros((pad,), jnp.int32)])
    ewb = jnp.concatenate(
        [jax.lax.bitcast_convert_type(edge_weight.astype(jnp.float32),
                                      jnp.int32),
         jnp.zeros((pad,), jnp.int32)])
    e3 = jnp.stack(
        [src.reshape(NW * NWIN, C), dst.reshape(NW * NWIN, C),
         ewb.reshape(NW * NWIN, C)], axis=1)

    hwp1 = _encode(x, W_enc, b_enc, W1)
    p1 = _sc_edge_pass(hwp1, e3)
    hwp2 = _mid(p1, b1, W2)
    p2 = _sc_edge_pass(hwp2, e3)
    return _decode(p2, b2, W_dec, b_dec)


# speedup vs baseline: 1.0500x; 1.0500x over previous
import dataclasses
import functools

import jax
import jax.numpy as jnp
from jax import lax
from jax.experimental import pallas as pl
from jax.experimental.pallas import tpu as pltpu
from jax.experimental.pallas import tpu_sc as plsc

N = 10000
D = 128
DP = D // 2
E = 320000
NCLS = 40

NC = 2
NS = 16
NW = NC * NS
L = 16

EPAD = 327680
EPT = EPAD // NW
C = 80
NWIN = EPT // C
RB = 4
SB = 2
IB = 8
NPAD = 10240
RPT = NPAD // NS


def _pack_tc(hw):
    u = jax.lax.bitcast_convert_type(
        hw.astype(jnp.bfloat16), jnp.uint16).astype(jnp.uint32)
    pk = [u[:, c * 32:c * 32 + 16] | (u[:, c * 32 + 16:c * 32 + 32] << 16)
          for c in range(D // 32)]
    return jnp.concatenate(pk, axis=1).astype(jnp.int32)


def _encode(x, W_enc, b_enc, W1):
    def body(x_ref, we_ref, be_ref, w1_ref, o_ref):
        h = jnp.tanh(
            jnp.dot(x_ref[...], we_ref[...], preferred_element_type=jnp.float32)
            + be_ref[...]
        )
        o_ref[...] = _pack_tc(
            jnp.dot(h, w1_ref[...], preferred_element_type=jnp.float32))

    return pl.pallas_call(
        body,
        out_shape=jax.ShapeDtypeStruct((N, DP), jnp.int32),
    )(x, W_enc, b_enc.reshape(1, D), W1)


def _mid(parts, b, W):
    def body(p_ref, b_ref, w_ref, o_ref):
        h = jnp.tanh(p_ref[0, :N, :] + p_ref[1, :N, :] + b_ref[...])
        o_ref[...] = _pack_tc(
            jnp.dot(h, w_ref[...], preferred_element_type=jnp.float32))

    return pl.pallas_call(
        body,
        out_shape=jax.ShapeDtypeStruct((N, DP), jnp.int32),
    )(parts, b.reshape(1, D), W)


def _decode(parts, b2, W_dec, b_dec):
    def body(p_ref, b2_ref, wd_ref, bd_ref, o_ref):
        h = jnp.tanh(p_ref[0, :N, :] + p_ref[1, :N, :] + b2_ref[...])
        o_ref[...] = (
            jnp.dot(h, wd_ref[...], preferred_element_type=jnp.float32)
            + bd_ref[...]
        )

    return pl.pallas_call(
        body,
        out_shape=jax.ShapeDtypeStruct((N, NCLS), jnp.float32),
    )(parts, b2.reshape(1, D), W_dec, b_dec.reshape(1, NCLS))


def _sc_edge_pass(hwp, e3, ew):
    mesh = plsc.VectorSubcoreMesh(core_axis_name="c", subcore_axis_name="s")
    cp = pltpu.CompilerParams()
    if "needs_layout_passes" in pltpu.CompilerParams.__dataclass_fields__:
        cp = dataclasses.replace(cp, needs_layout_passes=False)
    if "use_tc_tiling_on_sc" in pltpu.CompilerParams.__dataclass_fields__:
        cp = dataclasses.replace(cp, use_tc_tiling_on_sc=False)

    @functools.partial(
        pl.kernel,
        mesh=mesh,
        compiler_params=cp,
        out_type=jax.ShapeDtypeStruct((NC, NPAD, D), jnp.float32),
        scratch_types=(
            [pltpu.VMEM((2, C), jnp.int32) for _ in range(IB)]
            + [pltpu.VMEM((C,), jnp.float32) for _ in range(IB)]
            + [pltpu.VMEM((C, DP), jnp.int32) for _ in range(RB)]
            + [pltpu.VMEM((C, D), jnp.float32) for _ in range(SB)]
            + [pltpu.VMEM_SHARED((NPAD, D), jnp.float32)]
            + [pltpu.SemaphoreType.DMA for _ in range(IB + RB + SB)]
        ),
    )
    def k(hw_hbm, e3_hbm, ew_hbm, out_hbm, *refs):
        eib = refs[0:IB]
        ewb = refs[IB:2 * IB]
        rows = refs[2 * IB:2 * IB + RB]
        stage = refs[2 * IB + RB:2 * IB + RB + SB]
        acc_sh = refs[2 * IB + RB + SB]
        sems = refs[2 * IB + RB + SB + 1:]
        isem = sems[0:IB]
        gsem = sems[IB:IB + RB]
        ssem = sems[IB + RB:IB + RB + SB]

        cid = lax.axis_index("c")
        sid = lax.axis_index("s")
        wbase = (cid * NS + sid) * NWIN

        def idx_issue(wi, j):
            pltpu.async_copy(e3_hbm.at[wbase + wi], eib[j], isem[j])
            pltpu.async_copy(
                ew_hbm.at[pl.ds((wbase + wi) * C, C)], ewb[j], isem[j])

        def idx_wait(wi, j):
            pltpu.make_async_copy(
                e3_hbm.at[wbase + wi], eib[j], isem[j]).wait()
            pltpu.make_async_copy(
                ew_hbm.at[pl.ds((wbase + wi) * C, C)], ewb[j], isem[j]).wait()

        def gather_issue(j, b):
            pltpu.async_copy(hw_hbm.at[eib[j].at[0]], rows[b], gsem[b])

        def gather_wait(j, b):
            pltpu.make_async_copy(
                hw_hbm.at[eib[j].at[0]], rows[b], gsem[b]).wait()

        def scatter_issue(j, sb):
            pltpu.async_copy(stage[sb], acc_sh.at[eib[j].at[1]], ssem[sb],
                             add=True)

        def scatter_wait(j, sb):
            pltpu.make_async_copy(
                stage[sb], acc_sh.at[eib[j].at[1]], ssem[sb]).wait()

        for j in range(5):
            idx_issue(j, j)
        for j in range(3):
            idx_wait(j, j)
            gather_issue(j, j)

        @pl.loop(0, C)
        def _(r):
            for cc in range(D // L):
                stage[0][r, pl.ds(cc * L, L)] = jnp.zeros((L,), jnp.float32)
        for t in range(RPT // C):
            pltpu.sync_copy(stage[0],
                            acc_sh.at[pl.ds(sid * RPT + t * C, C)])
        plsc.subcore_barrier()

        @pl.loop(0, NWIN, step=IB)
        def _(w):
            for b in range(IB):
                wi = w + b
                rb = b % RB
                rb3 = (b + 3) % RB
                sb = b % SB
                j3 = (b + 3) % IB
                j5 = (b + 5) % IB

                @pl.when(wi + 5 < NWIN)
                def _():
                    idx_issue(wi + 5, j5)

                @pl.when(wi + 3 < NWIN)
                def _():
                    idx_wait(wi + 3, j3)
                    gather_issue(j3, rb3)

                gather_wait(b, rb)

                @pl.when(wi >= SB)
                def _():
                    scatter_wait((b - SB) % IB, sb)

                @plsc.parallel_loop(0, C, unroll=2)
                def _(r):
                    wvec = plsc.load_gather(
                        ewb[b], [jnp.full((L,), r, jnp.int32)])
                    for cc in range(D // 32):
                        pk = rows[rb][r, pl.ds(cc * 16, 16)]
                        bfv = plsc.bitcast(pk, jnp.bfloat16)
                        lo, hi = plsc.unpack(
                            bfv, format=plsc.PackFormat.INTERLEAVED)
                        stage[sb][r, pl.ds(cc * 32, L)] = lo * wvec
                        stage[sb][r, pl.ds(cc * 32 + L, L)] = hi * wvec

                scatter_issue(b, sb)

        for s in range(SB):
            scatter_wait((NWIN - SB + s) % IB, (NWIN - SB + s) % SB)

        plsc.subcore_barrier()
        pltpu.sync_copy(acc_sh.at[pl.ds(sid * RPT, RPT)],
                        out_hbm.at[cid, pl.ds(sid * RPT, RPT)])

    return k(hwp, e3, ew)


def kernel(x, edge_index, edge_weight, W_enc, b_enc, W1, b1, W2, b2, W_dec, b_dec):
    pad = EPAD - E
    src = jnp.concatenate(
        [edge_index[0].astype(jnp.int32), jnp.zeros((pad,), jnp.int32)])
    dst = jnp.concatenate(
        [edge_index[1].astype(jnp.int32), jnp.zeros((pad,), jnp.int32)])
    ew = jnp.concatenate(
        [edge_weight.astype(jnp.float32), jnp.zeros((pad,), jnp.float32)])
    e3 = jnp.stack(
        [src.reshape(NW * NWIN, C), dst.reshape(NW * NWIN, C)],
        axis=1)

    hwp1 = _encode(x, W_enc, b_enc, W1)
    p1 = _sc_edge_pass(hwp1, e3, ew)
    hwp2 = _mid(p1, b1, W2)
    p2 = _sc_edge_pass(hwp2, e3, ew)
    return _decode(p2, b2, W_dec, b_dec)


# speedup vs baseline: 1.1079x; 1.0552x over previous
import dataclasses
import functools

import jax
import jax.numpy as jnp
from jax import lax
from jax.experimental import pallas as pl
from jax.experimental.pallas import tpu as pltpu
from jax.experimental.pallas import tpu_sc as plsc

N = 10000
D = 128
DP = D // 2
E = 320000
NCLS = 40

NC = 2
NS = 16
NW = NC * NS
L = 16

EPAD = 327680
EPT = EPAD // NW
C = 80
NWIN = EPT // C
WA = 168
WB = 256 - WA
RB = 4
SB = 2
IB = 8
NPAD = 10240
RPT = NPAD // NS


def _pack_tc(hw):
    u = jax.lax.bitcast_convert_type(
        hw.astype(jnp.bfloat16), jnp.uint16).astype(jnp.uint32)
    pk = [u[:, c * 32:c * 32 + 16] | (u[:, c * 32 + 16:c * 32 + 32] << 16)
          for c in range(D // 32)]
    return jnp.concatenate(pk, axis=1).astype(jnp.int32)


def _encode(x, W_enc, b_enc, W1):
    def body(x_ref, we_ref, be_ref, w1_ref, o_ref):
        h = jnp.tanh(
            jnp.dot(x_ref[...], we_ref[...], preferred_element_type=jnp.float32)
            + be_ref[...]
        )
        o_ref[...] = _pack_tc(
            jnp.dot(h, w1_ref[...], preferred_element_type=jnp.float32))

    return pl.pallas_call(
        body,
        out_shape=jax.ShapeDtypeStruct((N, DP), jnp.int32),
    )(x, W_enc, b_enc.reshape(1, D), W1)


def _mid(parts, b, W):
    def body(p_ref, b_ref, w_ref, o_ref):
        h = jnp.tanh(p_ref[0, :N, :] + p_ref[1, :N, :] + b_ref[...])
        o_ref[...] = _pack_tc(
            jnp.dot(h, w_ref[...], preferred_element_type=jnp.float32))

    return pl.pallas_call(
        body,
        out_shape=jax.ShapeDtypeStruct((N, DP), jnp.int32),
    )(parts, b.reshape(1, D), W)


def _decode(parts, b2, W_dec, b_dec):
    def body(p_ref, b2_ref, wd_ref, bd_ref, o_ref):
        h = jnp.tanh(p_ref[0, :N, :] + p_ref[1, :N, :] + b2_ref[...])
        o_ref[...] = (
            jnp.dot(h, wd_ref[...], preferred_element_type=jnp.float32)
            + bd_ref[...]
        )

    return pl.pallas_call(
        body,
        out_shape=jax.ShapeDtypeStruct((N, NCLS), jnp.float32),
    )(parts, b2.reshape(1, D), W_dec, b_dec.reshape(1, NCLS))


def _sc_edge_pass(hwp, e3, ew):
    mesh = plsc.VectorSubcoreMesh(core_axis_name="c", subcore_axis_name="s")
    cp = pltpu.CompilerParams()
    if "needs_layout_passes" in pltpu.CompilerParams.__dataclass_fields__:
        cp = dataclasses.replace(cp, needs_layout_passes=False)
    if "use_tc_tiling_on_sc" in pltpu.CompilerParams.__dataclass_fields__:
        cp = dataclasses.replace(cp, use_tc_tiling_on_sc=False)

    @functools.partial(
        pl.kernel,
        mesh=mesh,
        compiler_params=cp,
        out_type=jax.ShapeDtypeStruct((NC, NPAD, D), jnp.float32),
        scratch_types=(
            [pltpu.VMEM((2, C), jnp.int32) for _ in range(IB)]
            + [pltpu.VMEM((C,), jnp.float32) for _ in range(IB)]
            + [pltpu.VMEM((C, DP), jnp.int32) for _ in range(RB)]
            + [pltpu.VMEM((C, D), jnp.float32) for _ in range(SB)]
            + [pltpu.VMEM_SHARED((NPAD, D), jnp.float32)]
            + [pltpu.SemaphoreType.DMA for _ in range(IB + RB + SB)]
        ),
    )
    def k(hw_hbm, e3_hbm, ew_hbm, out_hbm, *refs):
        eib = refs[0:IB]
        ewb = refs[IB:2 * IB]
        rows = refs[2 * IB:2 * IB + RB]
        stage = refs[2 * IB + RB:2 * IB + RB + SB]
        acc_sh = refs[2 * IB + RB + SB]
        sems = refs[2 * IB + RB + SB + 1:]
        isem = sems[0:IB]
        gsem = sems[IB:IB + RB]
        ssem = sems[IB + RB:IB + RB + SB]

        cid = lax.axis_index("c")
        sid = lax.axis_index("s")
        wbase = jnp.where(cid == 0, sid * WA, NS * WA + sid * WB)
        wn = jnp.where(cid == 0, WA, WB)

        def idx_issue(wi, j):
            pltpu.async_copy(e3_hbm.at[wbase + wi], eib[j], isem[j])
            pltpu.async_copy(
                ew_hbm.at[pl.ds((wbase + wi) * C, C)], ewb[j], isem[j])

        def idx_wait(wi, j):
            pltpu.make_async_copy(
                e3_hbm.at[wbase + wi], eib[j], isem[j]).wait()
            pltpu.make_async_copy(
                ew_hbm.at[pl.ds((wbase + wi) * C, C)], ewb[j], isem[j]).wait()

        def gather_issue(j, b):
            pltpu.async_copy(hw_hbm.at[eib[j].at[0]], rows[b], gsem[b])

        def gather_wait(j, b):
            pltpu.make_async_copy(
                hw_hbm.at[eib[j].at[0]], rows[b], gsem[b]).wait()

        def scatter_issue(j, sb):
            pltpu.async_copy(stage[sb], acc_sh.at[eib[j].at[1]], ssem[sb],
                             add=True)

        def scatter_wait(j, sb):
            pltpu.make_async_copy(
                stage[sb], acc_sh.at[eib[j].at[1]], ssem[sb]).wait()

        for j in range(5):
            idx_issue(j, j)
        for j in range(3):
            idx_wait(j, j)
            gather_issue(j, j)

        @pl.loop(0, C)
        def _(r):
            for cc in range(D // L):
                stage[0][r, pl.ds(cc * L, L)] = jnp.zeros((L,), jnp.float32)
        for t in range(RPT // C):
            pltpu.sync_copy(stage[0],
                            acc_sh.at[pl.ds(sid * RPT + t * C, C)])
        plsc.subcore_barrier()

        @pl.loop(0, wn, step=IB)
        def _(w):
            for b in range(IB):
                wi = w + b
                rb = b % RB
                rb3 = (b + 3) % RB
                sb = b % SB
                j3 = (b + 3) % IB
                j5 = (b + 5) % IB

                @pl.when(wi + 5 < wn)
                def _():
                    idx_issue(wi + 5, j5)

                @pl.when(wi + 3 < wn)
                def _():
                    idx_wait(wi + 3, j3)
                    gather_issue(j3, rb3)

                gather_wait(b, rb)

                @pl.when(wi >= SB)
                def _():
                    scatter_wait((b - SB) % IB, sb)

                @plsc.parallel_loop(0, C, unroll=2)
                def _(r):
                    wvec = plsc.load_gather(
                        ewb[b], [jnp.full((L,), r, jnp.int32)])
                    for cc in range(D // 32):
                        pk = rows[rb][r, pl.ds(cc * 16, 16)]
                        bfv = plsc.bitcast(pk, jnp.bfloat16)
                        lo, hi = plsc.unpack(
                            bfv, format=plsc.PackFormat.INTERLEAVED)
                        stage[sb][r, pl.ds(cc * 32, L)] = lo * wvec
                        stage[sb][r, pl.ds(cc * 32 + L, L)] = hi * wvec

                scatter_issue(b, sb)

        for s in range(SB):
            scatter_wait((-SB + s) % IB, (-SB + s) % SB)

        plsc.subcore_barrier()
        pltpu.sync_copy(acc_sh.at[pl.ds(sid * RPT, RPT)],
                        out_hbm.at[cid, pl.ds(sid * RPT, RPT)])

    return k(hwp, e3, ew)


def kernel(x, edge_index, edge_weight, W_enc, b_enc, W1, b1, W2, b2, W_dec, b_dec):
    pad = EPAD - E
    src = jnp.concatenate(
        [edge_index[0].astype(jnp.int32), jnp.zeros((pad,), jnp.int32)])
    dst = jnp.concatenate(
        [edge_index[1].astype(jnp.int32), jnp.zeros((pad,), jnp.int32)])
    ew = jnp.concatenate(
        [edge_weight.astype(jnp.float32), jnp.zeros((pad,), jnp.float32)])
    e3 = jnp.stack(
        [src.reshape(NW * NWIN, C), dst.reshape(NW * NWIN, C)],
        axis=1)

    hwp1 = _encode(x, W_enc, b_enc, W1)
    p1 = _sc_edge_pass(hwp1, e3, ew)
    hwp2 = _mid(p1, b1, W2)
    p2 = _sc_edge_pass(hwp2, e3, ew)
    return _decode(p2, b2, W_dec, b_dec)
